# baseline (speedup 1.0000x reference)
import functools

import jax
import jax.numpy as jnp
from jax import lax
from jax.experimental import pallas as pl
from jax.experimental.pallas import tpu as pltpu
from jax.experimental.pallas import tpu_sc as plsc

_VOCAB = 100000
_EMB = 64
_LIN1 = 256
_LIN2 = 64
_PAD = 128

_VOCAB_BLOCK = 10000
_GATHER_WINDOW = 256
_IDX_ROWS = 4


def _fold_body(cb_ref, w1_ref, w2_ref, b1_ref, b2_ref, out_ref):
    w1 = w1_ref[...]
    w2 = w2_ref[...]
    m = lax.dot_general(w2, w1, (((1,), (0,)), ((), ())),
                        preferred_element_type=jnp.float32)
    cb = cb_ref[...]
    acc = lax.dot_general(cb, m, (((1,), (1,)), ((), ())),
                          preferred_element_type=jnp.float32)
    c = lax.dot_general(b1_ref[...], w2, (((1,), (1,)), ((), ())),
                        preferred_element_type=jnp.float32)
    out_ref[:, : _LIN2] = acc + c + b2_ref[...]
    out_ref[:, _LIN2:] = jnp.zeros_like(out_ref[:, _LIN2:])


def _fold_table(codebook, W1, b1, W2, b2):
    return pl.pallas_call(
        _fold_body,
        grid=(_VOCAB // _VOCAB_BLOCK,),
        in_specs=[
            pl.BlockSpec((_VOCAB_BLOCK, _EMB), lambda i: (i, 0)),
            pl.BlockSpec((_LIN1, _EMB), lambda i: (0, 0)),
            pl.BlockSpec((_LIN2, _LIN1), lambda i: (0, 0)),
            pl.BlockSpec((1, _LIN1), lambda i: (0, 0)),
            pl.BlockSpec((1, _LIN2), lambda i: (0, 0)),
        ],
        out_specs=pl.BlockSpec((_VOCAB_BLOCK, _PAD), lambda i: (i, 0)),
        out_shape=jax.ShapeDtypeStruct((_VOCAB, _PAD), jnp.float32),
    )(codebook, W1, W2, b1.reshape(1, _LIN1), b2.reshape(1, _LIN2))


def _make_gather(num_indices):
    mesh = plsc.VectorSubcoreMesh(core_axis_name="c", subcore_axis_name="s")
    idx_cols = num_indices // _IDX_ROWS
    chunks_per_row = idx_cols // _GATHER_WINDOW

    @functools.partial(
        pl.kernel,
        out_type=jax.ShapeDtypeStruct((num_indices, _PAD), jnp.float32),
        mesh=mesh,
        compiler_params=pltpu.CompilerParams(use_tc_tiling_on_sc=True),
    )
    def _gather(table_hbm, idx_hbm, out_hbm):
        def body(i_vmem, o_vmem):
            pltpu.sync_copy(table_hbm.at[i_vmem.at[0]], o_vmem)

        pltpu.emit_pipeline(
            body,
            grid=(_IDX_ROWS, chunks_per_row),
            in_specs=[pl.BlockSpec((1, _GATHER_WINDOW), lambda i, j: (i, j))],
            out_specs=[pl.BlockSpec((_GATHER_WINDOW, _PAD),
                                    lambda i, j: (i * chunks_per_row + j, 0))],
            core_axis_name=("c", "s"),
            dimension_semantics=(pltpu.PARALLEL, pltpu.PARALLEL),
        )(idx_hbm, out_hbm)

    return _gather


def _transpose_first_body(in_ref, out_ref):
    out_ref[...] = in_ref[:, : _LIN2].T


def _transpose_rest_body(alias_ref, in_ref, out_ref):
    del alias_ref
    out_ref[...] = in_ref[:, : _LIN2].T


def _transpose_first(y, B, L, L_total):
    return pl.pallas_call(
        _transpose_first_body,
        grid=(L,),
        in_specs=[pl.BlockSpec((B, _PAD), lambda i: (i, 0))],
        out_specs=pl.BlockSpec((_LIN2, B), lambda i: (i, 0)),
        out_shape=jax.ShapeDtypeStruct((L_total * _LIN2, B), jnp.float32),
    )(y)


def _transpose_rest(acc, y, B, L, l_off):
    return pl.pallas_call(
        _transpose_rest_body,
        grid=(L,),
        in_specs=[
            pl.BlockSpec(memory_space=pl.ANY),
            pl.BlockSpec((B, _PAD), lambda i: (i, 0)),
        ],
        out_specs=pl.BlockSpec((_LIN2, B), lambda i: (i + l_off, 0)),
        out_shape=jax.ShapeDtypeStruct(acc.shape, jnp.float32),
        input_output_aliases={0: 0},
    )(acc, y)


_CHUNKS = 8


def kernel(x, codebook, W1, b1, W2, b2):
    B, L = x.shape
    lc = L // _CHUNKS
    idx = x.T.astype(jnp.int32)
    fused = _fold_table(codebook, W1, b1, W2, b2)
    gather = _make_gather(lc * B)
    gs = [gather(fused,
                 idx[k * lc:(k + 1) * lc].reshape(_IDX_ROWS, lc * B // _IDX_ROWS))
          for k in range(_CHUNKS)]
    acc = _transpose_first(gs[0], B, lc, L)
    for k in range(1, _CHUNKS):
        acc = _transpose_rest(acc, gs[k], B, lc, k * lc)
    return acc.reshape(L, _LIN2, B).transpose(2, 0, 1)


# speedup vs baseline: 1.0534x; 1.0534x over previous
import functools

import jax
import jax.numpy as jnp
from jax import lax
from jax.experimental import pallas as pl
from jax.experimental.pallas import tpu as pltpu
from jax.experimental.pallas import tpu_sc as plsc

_VOCAB = 100000
_EMB = 64
_LIN1 = 256
_LIN2 = 64
_PAD = 128

_VOCAB_BLOCK = 10000
_GATHER_WINDOW = 256
_IDX_ROWS = 4


def _fold_body(cb_ref, w1_ref, w2_ref, b1_ref, b2_ref, out_ref):
    w1 = w1_ref[...]
    w2 = w2_ref[...]
    m = lax.dot_general(w2, w1, (((1,), (0,)), ((), ())),
                        preferred_element_type=jnp.float32)
    cb = cb_ref[...]
    acc = lax.dot_general(cb, m, (((1,), (1,)), ((), ())),
                          preferred_element_type=jnp.float32)
    c = lax.dot_general(b1_ref[...], w2, (((1,), (1,)), ((), ())),
                        preferred_element_type=jnp.float32)
    out_ref[:, : _LIN2] = acc + c + b2_ref[...]
    out_ref[:, _LIN2:] = jnp.zeros_like(out_ref[:, _LIN2:])


def _fold_table(codebook, W1, b1, W2, b2):
    return pl.pallas_call(
        _fold_body,
        grid=(_VOCAB // _VOCAB_BLOCK,),
        in_specs=[
            pl.BlockSpec((_VOCAB_BLOCK, _EMB), lambda i: (i, 0)),
            pl.BlockSpec((_LIN1, _EMB), lambda i: (0, 0)),
            pl.BlockSpec((_LIN2, _LIN1), lambda i: (0, 0)),
            pl.BlockSpec((1, _LIN1), lambda i: (0, 0)),
            pl.BlockSpec((1, _LIN2), lambda i: (0, 0)),
        ],
        out_specs=pl.BlockSpec((_VOCAB_BLOCK, _PAD), lambda i: (i, 0)),
        out_shape=jax.ShapeDtypeStruct((_VOCAB, _PAD), jnp.float32),
    )(codebook, W1, W2, b1.reshape(1, _LIN1), b2.reshape(1, _LIN2))


def _make_gather(num_indices):
    mesh = plsc.VectorSubcoreMesh(core_axis_name="c", subcore_axis_name="s")
    idx_cols = num_indices // _IDX_ROWS
    chunks_per_row = idx_cols // _GATHER_WINDOW

    @functools.partial(
        pl.kernel,
        out_type=jax.ShapeDtypeStruct((num_indices, _PAD), jnp.float32),
        mesh=mesh,
        compiler_params=pltpu.CompilerParams(use_tc_tiling_on_sc=True),
    )
    def _gather(table_hbm, idx_hbm, out_hbm):
        def body(i_vmem, o_vmem):
            pltpu.sync_copy(table_hbm.at[i_vmem.at[0]], o_vmem)

        pltpu.emit_pipeline(
            body,
            grid=(_IDX_ROWS, chunks_per_row),
            in_specs=[pl.BlockSpec((1, _GATHER_WINDOW), lambda i, j: (i, j))],
            out_specs=[pl.BlockSpec((_GATHER_WINDOW, _PAD),
                                    lambda i, j: (i * chunks_per_row + j, 0))],
            core_axis_name=("c", "s"),
            dimension_semantics=(pltpu.PARALLEL, pltpu.PARALLEL),
        )(idx_hbm, out_hbm)

    return _gather


def _transpose_first_body(in_ref, out_ref):
    out_ref[...] = in_ref[:, : _LIN2].T


def _transpose_rest_body(alias_ref, in_ref, out_ref):
    del alias_ref
    out_ref[...] = in_ref[:, : _LIN2].T


def _transpose_first(y, B, L, L_total):
    return pl.pallas_call(
        _transpose_first_body,
        grid=(L,),
        in_specs=[pl.BlockSpec((B, _PAD), lambda i: (i, 0))],
        out_specs=pl.BlockSpec((_LIN2, B), lambda i: (i, 0)),
        out_shape=jax.ShapeDtypeStruct((L_total * _LIN2, B), jnp.float32),
    )(y)


def _transpose_rest(acc, y, B, L, l_off):
    return pl.pallas_call(
        _transpose_rest_body,
        grid=(L,),
        in_specs=[
            pl.BlockSpec(memory_space=pl.ANY),
            pl.BlockSpec((B, _PAD), lambda i: (i, 0)),
        ],
        out_specs=pl.BlockSpec((_LIN2, B), lambda i: (i + l_off, 0)),
        out_shape=jax.ShapeDtypeStruct(acc.shape, jnp.float32),
        input_output_aliases={0: 0},
    )(acc, y)


_CHUNKS = 4


def kernel(x, codebook, W1, b1, W2, b2):
    B, L = x.shape
    lc = L // _CHUNKS
    idx = x.T.astype(jnp.int32)
    fused = _fold_table(codebook, W1, b1, W2, b2)
    gather = _make_gather(lc * B)
    gs = [gather(fused,
                 idx[k * lc:(k + 1) * lc].reshape(_IDX_ROWS, lc * B // _IDX_ROWS))
          for k in range(_CHUNKS)]
    acc = _transpose_first(gs[0], B, lc, L)
    for k in range(1, _CHUNKS):
        acc = _transpose_rest(acc, gs[k], B, lc, k * lc)
    return acc.reshape(L, _LIN2, B).transpose(2, 0, 1)
